# baseline (speedup 1.0000x reference)
import functools

import jax
import jax.numpy as jnp
from jax import lax
from jax.experimental import pallas as pl
from jax.experimental.pallas import tpu as pltpu
from jax.experimental.pallas import tpu_sc as plsc

NEG_INF = -1000000000.0
_N = 4097
_W = 4096


def _sc_body(x_hbm, out_hbm, b0, b1, b2):
    wid = lax.axis_index("s") * 2 + lax.axis_index("c")

    r0 = b0.at[0]
    r1 = b1.at[0]
    r2 = b2.at[0]

    def process(r):
        pltpu.sync_copy(x_hbm.at[0].at[pl.ds(r, 1), pl.ds(0, _W)], b0)
        pltpu.sync_copy(x_hbm.at[1].at[pl.ds(r, 1), pl.ds(0, _W)], b1)
        pltpu.sync_copy(x_hbm.at[2].at[pl.ds(r, 1), pl.ds(0, _W)], b2)

        def col_loop(j, carry):
            c = j * 16
            a0 = r0[pl.ds(c, 16)]
            a1 = r1[pl.ds(c, 16)]
            a2 = r2[pl.ds(c, 16)]
            m = jnp.maximum(jnp.maximum(a0, a1), a2)
            e0 = jnp.exp(a0 - m)
            e1 = jnp.exp(a1 - m)
            e2 = jnp.exp(a2 - m)
            inv = 1.0 / (e0 + e1 + e2)
            r0[pl.ds(c, 16)] = e0 * inv
            r1[pl.ds(c, 16)] = e1 * inv
            r2[pl.ds(c, 16)] = e2 * inv
            return carry

        lax.fori_loop(0, _W // 16, col_loop, 0)

        pltpu.sync_copy(b0, out_hbm.at[0].at[pl.ds(r, 1), pl.ds(0, _W)])
        pltpu.sync_copy(b1, out_hbm.at[1].at[pl.ds(r, 1), pl.ds(0, _W)])
        pltpu.sync_copy(b2, out_hbm.at[2].at[pl.ds(r, 1), pl.ds(0, _W)])

    def work_loop(k, carry):
        process(wid + 32 * k)
        return carry

    lax.fori_loop(0, 128, work_loop, 0)

    @pl.when(wid == 0)
    def _last_row():
        r = _N - 1
        pltpu.sync_copy(x_hbm.at[1].at[pl.ds(r, 1), pl.ds(0, _W)], b1)
        pltpu.sync_copy(x_hbm.at[2].at[pl.ds(r, 1), pl.ds(0, _W)], b2)

        def col_loop(j, carry):
            c = j * 16
            a1 = r1[pl.ds(c, 16)]
            a2 = r2[pl.ds(c, 16)]
            m = jnp.maximum(a1, a2)
            e1 = jnp.exp(a1 - m)
            e2 = jnp.exp(a2 - m)
            inv = 1.0 / (e1 + e2)
            r0[pl.ds(c, 16)] = jnp.zeros((16,), jnp.float32)
            r1[pl.ds(c, 16)] = e1 * inv
            r2[pl.ds(c, 16)] = e2 * inv
            return carry

        lax.fori_loop(0, _W // 16, col_loop, 0)

        pltpu.sync_copy(b0, out_hbm.at[0].at[pl.ds(r, 1), pl.ds(0, _W)])
        pltpu.sync_copy(b1, out_hbm.at[1].at[pl.ds(r, 1), pl.ds(0, _W)])
        pltpu.sync_copy(b2, out_hbm.at[2].at[pl.ds(r, 1), pl.ds(0, _W)])


_FBR = 1024


def _fixup_block(x_ref, y_ref, o_ref, *, n, block_rows):
    i = pl.program_id(0)
    a0 = x_ref[0]
    a1 = x_ref[1]
    a2 = x_ref[2]
    row = jax.lax.broadcasted_iota(jnp.int32, a0.shape, 0) + i * block_rows
    col = jax.lax.broadcasted_iota(jnp.int32, a0.shape, 1) + (n - 1)
    a0 = jnp.where(row == n - 1, NEG_INF, a0)
    a1 = jnp.where(col == n - 1, NEG_INF, a1)
    m = jnp.maximum(jnp.maximum(a0, a1), a2)
    e0 = jnp.exp(a0 - m)
    e1 = jnp.exp(a1 - m)
    e2 = jnp.exp(a2 - m)
    inv = 1.0 / (e0 + e1 + e2)
    o_ref[0] = e0 * inv
    o_ref[1] = e1 * inv
    o_ref[2] = e2 * inv


def kernel(log_edge_flows):
    n = log_edge_flows.shape[0]
    x = jnp.transpose(log_edge_flows, (2, 0, 1))

    mesh = plsc.VectorSubcoreMesh(core_axis_name="c", subcore_axis_name="s")
    y = pl.kernel(
        _sc_body,
        mesh=mesh,
        out_type=jax.ShapeDtypeStruct((3, n, n), jnp.float32),
        scratch_types=[
            pltpu.VMEM((1, _W), jnp.float32),
            pltpu.VMEM((1, _W), jnp.float32),
            pltpu.VMEM((1, _W), jnp.float32),
        ],
    )(x)

    strip = pl.BlockSpec((3, _FBR, 128), lambda i: (0, i, (n - 1) // 128))
    out = pl.pallas_call(
        functools.partial(_fixup_block, n=n, block_rows=_FBR),
        grid=(pl.cdiv(n, _FBR),),
        in_specs=[strip, strip],
        out_specs=strip,
        out_shape=jax.ShapeDtypeStruct((3, n, n), jnp.float32),
        input_output_aliases={1: 0},
    )(x, y)
    return jnp.transpose(out, (1, 2, 0))


# speedup vs baseline: 9.2588x; 9.2588x over previous
import functools

import jax
import jax.numpy as jnp
from jax.experimental import pallas as pl

NEG_INF = -1000000000.0
_BR = 224


def _softmax3_block(x_ref, o_ref, *, n, block_rows):
    i = pl.program_id(0)
    a0 = x_ref[0]
    a1 = x_ref[1]
    a2 = x_ref[2]
    row = jax.lax.broadcasted_iota(jnp.int32, a0.shape, 0) + i * block_rows
    col = jax.lax.broadcasted_iota(jnp.int32, a0.shape, 1)
    a0 = jnp.where(row == n - 1, NEG_INF, a0)
    a1 = jnp.where(col == n - 1, NEG_INF, a1)
    m = jnp.maximum(jnp.maximum(a0, a1), a2)
    e0 = jnp.exp(a0 - m)
    e1 = jnp.exp(a1 - m)
    e2 = jnp.exp(a2 - m)
    inv = 1.0 / (e0 + e1 + e2)
    o_ref[0] = e0 * inv
    o_ref[1] = e1 * inv
    o_ref[2] = e2 * inv


def kernel(log_edge_flows):
    n = log_edge_flows.shape[0]
    x = jnp.transpose(log_edge_flows, (2, 0, 1))
    grid = (pl.cdiv(n, _BR),)
    out = pl.pallas_call(
        functools.partial(_softmax3_block, n=n, block_rows=_BR),
        grid=grid,
        in_specs=[pl.BlockSpec((3, _BR, n), lambda i: (0, i, 0))],
        out_specs=pl.BlockSpec((3, _BR, n), lambda i: (0, i, 0)),
        out_shape=jax.ShapeDtypeStruct((3, n, n), jnp.float32),
    )(x)
    return jnp.transpose(out, (1, 2, 0))
